# Initial kernel scaffold; baseline (speedup 1.0000x reference)
import jax
import jax.numpy as jnp
from jax.experimental import pallas as pl


def kernel(x, base_W, base_b, router_W, A, Bm):
    raise NotImplementedError("write your pallas kernel here")


# speedup vs baseline: 4.8428x; 4.8428x over previous
import functools

import jax
import jax.numpy as jnp
from jax.experimental import pallas as pl

B_, S_, D_ = 4, 8192, 768
E_, R_, K_ = 8, 4, 2


def _fused_body(x_ref, wb_ref, rw_ref, a_ref, bm_ref, bias_ref, out_ref):
    xt = x_ref[...]
    base = jnp.dot(xt, wb_ref[...], preferred_element_type=jnp.float32)
    logits = jnp.dot(xt, rw_ref[...], preferred_element_type=jnp.float32)
    h = jnp.dot(xt, a_ref[...], preferred_element_type=jnp.float32)

    mx = jnp.max(logits, axis=1, keepdims=True)
    ex = jnp.exp(logits - mx)
    p = ex / jnp.sum(ex, axis=1, keepdims=True)

    idx = jax.lax.broadcasted_iota(jnp.int32, p.shape, 1)
    m1 = jnp.max(p, axis=1, keepdims=True)
    i1 = jnp.min(jnp.where(p == m1, idx, E_), axis=1, keepdims=True)
    pm = jnp.where(idx == i1, -jnp.inf, p)
    m2 = jnp.max(pm, axis=1, keepdims=True)
    i2 = jnp.min(jnp.where(pm == m2, idx, E_), axis=1, keepdims=True)
    denom = m1 + m2 + 1e-6
    g = (jnp.where(idx == i1, m1, 0.0) + jnp.where(idx == i2, m2, 0.0)) / denom

    ei = jax.lax.broadcasted_iota(jnp.int32, (E_, E_ * R_), 0)
    ej = jax.lax.broadcasted_iota(jnp.int32, (E_, E_ * R_), 1)
    expand = jnp.where(ej // R_ == ei, 1.0, 0.0)
    gexp = jnp.dot(g, expand, preferred_element_type=jnp.float32)

    y = jnp.dot(h * gexp, bm_ref[...], preferred_element_type=jnp.float32)
    out_ref[...] = base + y + bias_ref[...]


@functools.partial(jax.jit, static_argnames=("tile_m",))
def _run(x2d, wbT, rwT, aT, bmT, bias2d, tile_m=512):
    m = x2d.shape[0]
    grid = (m // tile_m,)
    return pl.pallas_call(
        _fused_body,
        grid=grid,
        in_specs=[
            pl.BlockSpec((tile_m, D_), lambda i: (i, 0)),
            pl.BlockSpec((D_, D_), lambda i: (0, 0)),
            pl.BlockSpec((D_, E_), lambda i: (0, 0)),
            pl.BlockSpec((D_, E_ * R_), lambda i: (0, 0)),
            pl.BlockSpec((E_ * R_, D_), lambda i: (0, 0)),
            pl.BlockSpec((1, D_), lambda i: (0, 0)),
        ],
        out_specs=pl.BlockSpec((tile_m, D_), lambda i: (i, 0)),
        out_shape=jax.ShapeDtypeStruct((m, D_), jnp.float32),
    )(x2d, wbT, rwT, aT, bmT, bias2d)


def kernel(x, base_W, base_b, router_W, A, Bm):
    b, s, d = x.shape
    x2d = x.reshape(b * s, d)
    wbT = base_W.T
    rwT = router_W.T
    aT = A.reshape(E_ * R_, D_).T
    bmT = jnp.transpose(Bm, (0, 2, 1)).reshape(E_ * R_, D_)
    bias2d = base_b.reshape(1, D_)
    out = _run(x2d, wbT, rwT, aT, bmT, bias2d)
    return out.reshape(b, s, d)


# speedup vs baseline: 4.8512x; 1.0017x over previous
import functools

import jax
import jax.numpy as jnp
from jax.experimental import pallas as pl

B_, S_, D_ = 4, 8192, 768
E_, R_, K_ = 8, 4, 2


def _fused_body(x_ref, wb_ref, rw_ref, a_ref, bm_ref, bias_ref, out_ref):
    xt = x_ref[...]
    xb = xt.astype(jnp.bfloat16)
    base = jnp.dot(xb, wb_ref[...], preferred_element_type=jnp.float32)
    logits = jnp.dot(xt, rw_ref[...], preferred_element_type=jnp.float32)
    h = jnp.dot(xb, a_ref[...], preferred_element_type=jnp.float32)

    mx = jnp.max(logits, axis=1, keepdims=True)
    ex = jnp.exp(logits - mx)
    p = ex / jnp.sum(ex, axis=1, keepdims=True)

    idx = jax.lax.broadcasted_iota(jnp.int32, p.shape, 1)
    m1 = jnp.max(p, axis=1, keepdims=True)
    i1 = jnp.min(jnp.where(p == m1, idx, E_), axis=1, keepdims=True)
    pm = jnp.where(idx == i1, -jnp.inf, p)
    m2 = jnp.max(pm, axis=1, keepdims=True)
    i2 = jnp.min(jnp.where(pm == m2, idx, E_), axis=1, keepdims=True)
    denom = m1 + m2 + 1e-6
    g = (jnp.where(idx == i1, m1, 0.0) + jnp.where(idx == i2, m2, 0.0)) / denom

    ei = jax.lax.broadcasted_iota(jnp.int32, (E_, E_ * R_), 0)
    ej = jax.lax.broadcasted_iota(jnp.int32, (E_, E_ * R_), 1)
    expand = jnp.where(ej // R_ == ei, 1.0, 0.0)
    gexp = jnp.dot(g, expand, preferred_element_type=jnp.float32)

    y = jnp.dot((h * gexp).astype(jnp.bfloat16), bm_ref[...],
                preferred_element_type=jnp.float32)
    out_ref[...] = base + y + bias_ref[...]


@functools.partial(jax.jit, static_argnames=("tile_m",))
def _run(x2d, wbT, rwT, aT, bmT, bias2d, tile_m=512):
    m = x2d.shape[0]
    grid = (m // tile_m,)
    return pl.pallas_call(
        _fused_body,
        grid=grid,
        in_specs=[
            pl.BlockSpec((tile_m, D_), lambda i: (i, 0)),
            pl.BlockSpec((D_, D_), lambda i: (0, 0)),
            pl.BlockSpec((D_, E_), lambda i: (0, 0)),
            pl.BlockSpec((D_, E_ * R_), lambda i: (0, 0)),
            pl.BlockSpec((E_ * R_, D_), lambda i: (0, 0)),
            pl.BlockSpec((1, D_), lambda i: (0, 0)),
        ],
        out_specs=pl.BlockSpec((tile_m, D_), lambda i: (i, 0)),
        out_shape=jax.ShapeDtypeStruct((m, D_), jnp.float32),
    )(x2d, wbT, rwT, aT, bmT, bias2d)


def kernel(x, base_W, base_b, router_W, A, Bm):
    b, s, d = x.shape
    x2d = x.reshape(b * s, d)
    wbT = base_W.T.astype(jnp.bfloat16)
    rwT = router_W.T
    aT = A.reshape(E_ * R_, D_).T.astype(jnp.bfloat16)
    bmT = jnp.transpose(Bm, (0, 2, 1)).reshape(E_ * R_, D_).astype(jnp.bfloat16)
    bias2d = base_b.reshape(1, D_)
    out = _run(x2d, wbT, rwT, aT, bmT, bias2d)
    return out.reshape(b, s, d)


# speedup vs baseline: 6.5243x; 1.3449x over previous
import functools

import jax
import jax.numpy as jnp
from jax.experimental import pallas as pl

B_, S_, D_ = 4, 8192, 768
E_, R_, K_ = 8, 4, 2


def _fused_body(x_ref, wb_ref, rw_ref, a_ref, bm_ref, bias_ref, out_ref):
    xt = x_ref[...]
    xb = xt.astype(jnp.bfloat16)
    base = jnp.dot(xb, wb_ref[...], preferred_element_type=jnp.float32)
    logits_t = jax.lax.dot_general(
        rw_ref[...], xt, (((1,), (1,)), ((), ())),
        preferred_element_type=jnp.float32)
    h = jnp.dot(xb, a_ref[...], preferred_element_type=jnp.float32)

    mx = jnp.max(logits_t, axis=0, keepdims=True)
    ex = jnp.exp(logits_t - mx)
    z = jnp.sum(ex, axis=0, keepdims=True)

    idx = jax.lax.broadcasted_iota(jnp.int32, ex.shape, 0)
    m1 = jnp.max(ex, axis=0, keepdims=True)
    i1 = jnp.min(jnp.where(ex == m1, idx, E_), axis=0, keepdims=True)
    pm = jnp.where(idx == i1, -jnp.inf, ex)
    m2 = jnp.max(pm, axis=0, keepdims=True)
    i2 = jnp.min(jnp.where(pm == m2, idx, E_), axis=0, keepdims=True)
    denom = m1 + m2 + 1e-6 * z
    g_t = (jnp.where(idx == i1, m1, 0.0) + jnp.where(idx == i2, m2, 0.0)) / denom
    g = jnp.transpose(g_t)

    ei = jax.lax.broadcasted_iota(jnp.int32, (E_, E_ * R_), 0)
    ej = jax.lax.broadcasted_iota(jnp.int32, (E_, E_ * R_), 1)
    expand = jnp.where(ej // R_ == ei, 1.0, 0.0)
    gexp = jnp.dot(g, expand, preferred_element_type=jnp.float32)

    y = jnp.dot((h * gexp).astype(jnp.bfloat16), bm_ref[...],
                preferred_element_type=jnp.float32)
    out_ref[...] = base + y + bias_ref[...]


@functools.partial(jax.jit, static_argnames=("tile_m",))
def _run(x2d, wbT, rwT, aT, bmT, bias2d, tile_m=512):
    m = x2d.shape[0]
    grid = (m // tile_m,)
    return pl.pallas_call(
        _fused_body,
        grid=grid,
        in_specs=[
            pl.BlockSpec((tile_m, D_), lambda i: (i, 0)),
            pl.BlockSpec((D_, D_), lambda i: (0, 0)),
            pl.BlockSpec((E_, D_), lambda i: (0, 0)),
            pl.BlockSpec((D_, E_ * R_), lambda i: (0, 0)),
            pl.BlockSpec((E_ * R_, D_), lambda i: (0, 0)),
            pl.BlockSpec((1, D_), lambda i: (0, 0)),
        ],
        out_specs=pl.BlockSpec((tile_m, D_), lambda i: (i, 0)),
        out_shape=jax.ShapeDtypeStruct((m, D_), jnp.float32),
    )(x2d, wbT, rwT, aT, bmT, bias2d)


def kernel(x, base_W, base_b, router_W, A, Bm):
    b, s, d = x.shape
    x2d = x.reshape(b * s, d)
    wbT = base_W.T.astype(jnp.bfloat16)
    rwT = router_W
    aT = A.reshape(E_ * R_, D_).T.astype(jnp.bfloat16)
    bmT = jnp.transpose(Bm, (0, 2, 1)).reshape(E_ * R_, D_).astype(jnp.bfloat16)
    bias2d = base_b.reshape(1, D_)
    out = _run(x2d, wbT, rwT, aT, bmT, bias2d)
    return out.reshape(b, s, d)


# speedup vs baseline: 6.7971x; 1.0418x over previous
import functools

import jax
import jax.numpy as jnp
from jax.experimental import pallas as pl

B_, S_, D_ = 4, 8192, 768
E_, R_, K_ = 8, 4, 2


def _fused_body(x_ref, wb_ref, rw_ref, a_ref, bm_ref, bias_ref, out_ref):
    xt = x_ref[...]
    xb = xt.astype(jnp.bfloat16)
    base = jnp.dot(xb, wb_ref[...], preferred_element_type=jnp.float32)
    logits_t = jax.lax.dot_general(
        rw_ref[...], xt, (((1,), (1,)), ((), ())),
        preferred_element_type=jnp.float32)
    h_t = jax.lax.dot_general(
        a_ref[...], xb, (((1,), (1,)), ((), ())),
        preferred_element_type=jnp.float32)

    mx = jnp.max(logits_t, axis=0, keepdims=True)
    ex = jnp.exp(logits_t - mx)
    z = jnp.sum(ex, axis=0, keepdims=True)

    idx = jax.lax.broadcasted_iota(jnp.int32, ex.shape, 0)
    m1 = jnp.max(ex, axis=0, keepdims=True)
    i1 = jnp.min(jnp.where(ex == m1, idx, E_), axis=0, keepdims=True)
    pm = jnp.where(idx == i1, -jnp.inf, ex)
    m2 = jnp.max(pm, axis=0, keepdims=True)
    i2 = jnp.min(jnp.where(pm == m2, idx, E_), axis=0, keepdims=True)
    denom = m1 + m2 + 1e-6 * z
    g_t = (jnp.where(idx == i1, m1, 0.0) + jnp.where(idx == i2, m2, 0.0)) / denom

    gexp_t = jnp.concatenate([g_t] * R_, axis=0)
    wh_t = (h_t * gexp_t).astype(jnp.bfloat16)
    y = jax.lax.dot_general(
        wh_t, bm_ref[...], (((0,), (0,)), ((), ())),
        preferred_element_type=jnp.float32)
    out_ref[...] = base + y + bias_ref[...]


@functools.partial(jax.jit, static_argnames=("tile_m",))
def _run(x2d, wbT, rwT, aT, bmT, bias2d, tile_m=512):
    m = x2d.shape[0]
    grid = (m // tile_m,)
    return pl.pallas_call(
        _fused_body,
        grid=grid,
        in_specs=[
            pl.BlockSpec((tile_m, D_), lambda i: (i, 0)),
            pl.BlockSpec((D_, D_), lambda i: (0, 0)),
            pl.BlockSpec((E_, D_), lambda i: (0, 0)),
            pl.BlockSpec((E_ * R_, D_), lambda i: (0, 0)),
            pl.BlockSpec((E_ * R_, D_), lambda i: (0, 0)),
            pl.BlockSpec((1, D_), lambda i: (0, 0)),
        ],
        out_specs=pl.BlockSpec((tile_m, D_), lambda i: (i, 0)),
        out_shape=jax.ShapeDtypeStruct((m, D_), jnp.float32),
    )(x2d, wbT, rwT, aT, bmT, bias2d)


def kernel(x, base_W, base_b, router_W, A, Bm):
    b, s, d = x.shape
    x2d = x.reshape(b * s, d)
    wbT = base_W.T.astype(jnp.bfloat16)
    rwT = router_W
    aT = jnp.transpose(A, (1, 0, 2)).reshape(R_ * E_, D_).astype(jnp.bfloat16)
    bmT = jnp.transpose(Bm, (2, 0, 1)).reshape(R_ * E_, D_).astype(jnp.bfloat16)
    bias2d = base_b.reshape(1, D_)
    out = _run(x2d, wbT, rwT, aT, bmT, bias2d)
    return out.reshape(b, s, d)


# speedup vs baseline: 8.3044x; 1.2217x over previous
import functools

import jax
import jax.numpy as jnp
from jax.experimental import pallas as pl

B_, S_, D_ = 4, 8192, 768
E_, R_, K_ = 8, 4, 2


def _fused_body(x_ref, wb_ref, rw_ref, a_ref, bm_ref, bias_ref, out_ref):
    xt = x_ref[...]
    xb = xt.astype(jnp.bfloat16)
    base = jnp.dot(xb, wb_ref[...], preferred_element_type=jnp.float32)
    logits_t = jax.lax.dot_general(
        rw_ref[...], xt, (((1,), (1,)), ((), ())),
        preferred_element_type=jnp.float32)
    h_t = jax.lax.dot_general(
        a_ref[...], xb, (((1,), (1,)), ((), ())),
        preferred_element_type=jnp.float32)

    mx = jnp.max(logits_t, axis=0, keepdims=True)
    ex = jnp.exp(logits_t - mx)
    z = jnp.sum(ex, axis=0, keepdims=True)

    idx = jax.lax.broadcasted_iota(jnp.int32, ex.shape, 0)
    m1 = jnp.max(ex, axis=0, keepdims=True)
    i1 = jnp.min(jnp.where(ex == m1, idx, E_), axis=0, keepdims=True)
    pm = jnp.where(idx == i1, -jnp.inf, ex)
    m2 = jnp.max(pm, axis=0, keepdims=True)
    i2 = jnp.min(jnp.where(pm == m2, idx, E_), axis=0, keepdims=True)
    denom = m1 + m2 + 1e-6 * z
    g_t = (jnp.where(idx == i1, m1, 0.0) + jnp.where(idx == i2, m2, 0.0)) / denom

    gexp_t = jnp.concatenate([g_t] * R_, axis=0)
    wh_t = (h_t * gexp_t).astype(jnp.bfloat16)
    y = jax.lax.dot_general(
        wh_t, bm_ref[...], (((0,), (0,)), ((), ())),
        preferred_element_type=jnp.float32)
    out_ref[...] = base + y + bias_ref[...]


@functools.partial(jax.jit, static_argnames=("tile_m",))
def _run(x2d, wbT, rwT, aT, bmT, bias2d, tile_m=1024):
    m = x2d.shape[0]
    grid = (m // tile_m,)
    return pl.pallas_call(
        _fused_body,
        grid=grid,
        in_specs=[
            pl.BlockSpec((tile_m, D_), lambda i: (i, 0)),
            pl.BlockSpec((D_, D_), lambda i: (0, 0)),
            pl.BlockSpec((E_, D_), lambda i: (0, 0)),
            pl.BlockSpec((E_ * R_, D_), lambda i: (0, 0)),
            pl.BlockSpec((E_ * R_, D_), lambda i: (0, 0)),
            pl.BlockSpec((1, D_), lambda i: (0, 0)),
        ],
        out_specs=pl.BlockSpec((tile_m, D_), lambda i: (i, 0)),
        out_shape=jax.ShapeDtypeStruct((m, D_), jnp.float32),
    )(x2d, wbT, rwT, aT, bmT, bias2d)


def kernel(x, base_W, base_b, router_W, A, Bm):
    b, s, d = x.shape
    x2d = x.reshape(b * s, d)
    wbT = base_W.T.astype(jnp.bfloat16)
    rwT = router_W
    aT = jnp.transpose(A, (1, 0, 2)).reshape(R_ * E_, D_).astype(jnp.bfloat16)
    bmT = jnp.transpose(Bm, (2, 0, 1)).reshape(R_ * E_, D_).astype(jnp.bfloat16)
    bias2d = base_b.reshape(1, D_)
    out = _run(x2d, wbT, rwT, aT, bmT, bias2d)
    return out.reshape(b, s, d)


# speedup vs baseline: 9.0018x; 1.0840x over previous
import functools

import jax
import jax.numpy as jnp
from jax.experimental import pallas as pl

B_, S_, D_ = 4, 8192, 768
E_, R_, K_ = 8, 4, 2


def _fused_body(x_ref, wb_ref, rw_ref, a_ref, bm_ref, bias_ref, out_ref):
    xt = x_ref[...]
    xb = xt.astype(jnp.bfloat16)
    base = jnp.dot(xb, wb_ref[...], preferred_element_type=jnp.float32)
    logits_t = jax.lax.dot_general(
        rw_ref[...], xt, (((1,), (1,)), ((), ())),
        preferred_element_type=jnp.float32)
    h_t = jax.lax.dot_general(
        a_ref[...], xb, (((1,), (1,)), ((), ())),
        preferred_element_type=jnp.float32)

    mx = jnp.max(logits_t, axis=0, keepdims=True)
    ex = jnp.exp(logits_t - mx)
    z = jnp.sum(ex, axis=0, keepdims=True)

    idx = jax.lax.broadcasted_iota(jnp.int32, ex.shape, 0)
    m1 = jnp.max(ex, axis=0, keepdims=True)
    i1 = jnp.min(jnp.where(ex == m1, idx, E_), axis=0, keepdims=True)
    pm = jnp.where(idx == i1, -jnp.inf, ex)
    m2 = jnp.max(pm, axis=0, keepdims=True)
    i2 = jnp.min(jnp.where(pm == m2, idx, E_), axis=0, keepdims=True)
    denom = m1 + m2 + 1e-6 * z
    g_t = (jnp.where(idx == i1, m1, 0.0) + jnp.where(idx == i2, m2, 0.0)) / denom

    gexp_t = jnp.concatenate([g_t] * R_, axis=0)
    wh_t = (h_t * gexp_t).astype(jnp.bfloat16)
    y = jax.lax.dot_general(
        wh_t, bm_ref[...], (((0,), (0,)), ((), ())),
        preferred_element_type=jnp.float32)
    out_ref[...] = base + y + bias_ref[...]


@functools.partial(jax.jit, static_argnames=("tile_m",))
def _run(x2d, wbT, rwT, aT, bmT, bias2d, tile_m=2048):
    m = x2d.shape[0]
    grid = (m // tile_m,)
    return pl.pallas_call(
        _fused_body,
        grid=grid,
        in_specs=[
            pl.BlockSpec((tile_m, D_), lambda i: (i, 0)),
            pl.BlockSpec((D_, D_), lambda i: (0, 0)),
            pl.BlockSpec((E_, D_), lambda i: (0, 0)),
            pl.BlockSpec((E_ * R_, D_), lambda i: (0, 0)),
            pl.BlockSpec((E_ * R_, D_), lambda i: (0, 0)),
            pl.BlockSpec((1, D_), lambda i: (0, 0)),
        ],
        out_specs=pl.BlockSpec((tile_m, D_), lambda i: (i, 0)),
        out_shape=jax.ShapeDtypeStruct((m, D_), jnp.float32),
    )(x2d, wbT, rwT, aT, bmT, bias2d)


def kernel(x, base_W, base_b, router_W, A, Bm):
    b, s, d = x.shape
    x2d = x.reshape(b * s, d)
    wbT = base_W.T.astype(jnp.bfloat16)
    rwT = router_W
    aT = jnp.transpose(A, (1, 0, 2)).reshape(R_ * E_, D_).astype(jnp.bfloat16)
    bmT = jnp.transpose(Bm, (2, 0, 1)).reshape(R_ * E_, D_).astype(jnp.bfloat16)
    bias2d = base_b.reshape(1, D_)
    out = _run(x2d, wbT, rwT, aT, bmT, bias2d)
    return out.reshape(b, s, d)
